# baseline (speedup 1.0000x reference)
import jax
import jax.numpy as jnp
from jax import lax
from jax.experimental import pallas as pl
from jax.experimental.pallas import tpu as pltpu
from jax.experimental.pallas import tpu_sc as plsc

N_GRAPHS = 256
N_PER = 64
EMB = 16
OUT = 64
E_EDGES = 262144
TOTAL_PAIRS = N_GRAPHS * N_PER * N_PER

HALF = TOTAL_PAIRS // 2
CHUNK = 8192
CH_SH = 13
NCH = HALF // CHUNK
NTILES = 16
EPT = E_EDGES // NTILES
SBLK = 4096
NBLK = EPT // SBLK
K = 128
CPT = CHUNK // NTILES
EID_BITS = 18

ROW_BLK = 8192


def _mm_body(x_ref, w_ref, d_ref, o_ref):
    o_ref[...] = lax.dot_general(
        x_ref[...], w_ref[...],
        dimension_numbers=(((1,), (1,)), ((), ())),
        preferred_element_type=jnp.float32) + d_ref[:, :OUT]


def _matmul_add(gm_val, W, delta):
    grid = TOTAL_PAIRS // ROW_BLK
    return pl.pallas_call(
        _mm_body,
        grid=(grid,),
        in_specs=[
            pl.BlockSpec((ROW_BLK, EMB), lambda i: (i, 0)),
            pl.BlockSpec((OUT, EMB), lambda i: (0, 0)),
            pl.BlockSpec((ROW_BLK, 2 * OUT), lambda i: (i, 0)),
        ],
        out_specs=pl.BlockSpec((ROW_BLK, OUT), lambda i: (i, 0)),
        out_shape=jax.ShapeDtypeStruct((TOTAL_PAIRS, OUT), jnp.float32),
    )(gm_val, W, delta)


def _sc_body(es_hbm, ed_hbm, ea_hbm, out_hbm,
             sst, sdt, packed,
             cntv, offv, nextv,
             gidx, sidx, grow, zbuf, chunk_sp):
    c = lax.axis_index("c")
    s = lax.axis_index("s")
    lo = c * HALF
    base_e = s * EPT
    iota16 = lax.iota(jnp.int32, 16)

    zero16 = jnp.zeros((16,), jnp.int32)
    for j in range(NCH // 16):
        cntv[pl.ds(j * 16, 16)] = zero16

    def p1_blk(t, carry):
        pltpu.sync_copy(es_hbm.at[pl.ds(base_e + t * SBLK, SBLK)], sst)
        pltpu.sync_copy(ed_hbm.at[pl.ds(base_e + t * SBLK, SBLK)], sdt)

        def p1(i, icarry):
            sv = sst[pl.ds(i * 16, 16)]
            dv = sdt[pl.ds(i * 16, 16)]
            p = sv * OUT + jnp.bitwise_and(dv, N_PER - 1)
            q = p - lo
            mine = (q >= 0) & (q < HALF)
            b = jnp.where(mine, lax.shift_right_logical(q, CH_SH), 0)
            cnt, last = plsc.scan_count(b, mask=mine)
            plsc.addupdate_scatter(cntv, [b], cnt, mask=last & mine)
            return icarry

        lax.fori_loop(0, SBLK // 16, p1, 0)
        return carry

    lax.fori_loop(0, NBLK, p1_blk, 0)

    carry = jnp.int32(0)
    for j in range(NCH // 16):
        v = cntv[pl.ds(j * 16, 16)]
        cs = plsc.cumsum(v)
        excl = cs - v + carry
        offv[pl.ds(j * 16, 16)] = excl
        nextv[pl.ds(j * 16, 16)] = excl
        carry = carry + jnp.sum(v)

    def p3_blk(t, carry):
        pltpu.sync_copy(es_hbm.at[pl.ds(base_e + t * SBLK, SBLK)], sst)
        pltpu.sync_copy(ed_hbm.at[pl.ds(base_e + t * SBLK, SBLK)], sdt)

        def p3(i, icarry):
            sv = sst[pl.ds(i * 16, 16)]
            dv = sdt[pl.ds(i * 16, 16)]
            p = sv * OUT + jnp.bitwise_and(dv, N_PER - 1)
            q = p - lo
            mine = (q >= 0) & (q < HALF)
            b = jnp.where(mine, lax.shift_right_logical(q, CH_SH), 0)
            cnt, last = plsc.scan_count(b, mask=mine)
            basev = plsc.load_gather(nextv, [b], mask=mine)
            dest = basev + cnt - 1
            posl = jnp.bitwise_and(q, CHUNK - 1)
            eid = base_e + t * SBLK + i * 16 + iota16
            val = jnp.bitwise_or(lax.shift_left(posl, EID_BITS), eid)
            plsc.store_scatter(packed, [dest], val, mask=mine)
            plsc.addupdate_scatter(nextv, [b], cnt, mask=last & mine)
            return icarry

        lax.fori_loop(0, SBLK // 16, p3, 0)
        return carry

    lax.fori_loop(0, NBLK, p3_blk, 0)

    zf32 = jnp.zeros((16,), jnp.float32)

    def zfill(i, carry):
        for j in range(8):
            zbuf[i, pl.ds(j * 16, 16)] = zf32
        return carry

    lax.fori_loop(0, 64, zfill, 0)

    def chunk_body(ch, carry):
        row0 = lo + ch * CHUNK
        myr = s * CPT
        for z in range(CPT // 64):
            pltpu.sync_copy(zbuf, chunk_sp.at[pl.ds(myr + z * 64, 64)])
        plsc.subcore_barrier()
        g = lax.shift_left(lax.shift_right_logical(ch, 4), 4)
        lane = jnp.bitwise_and(ch, 15)
        zi = jnp.zeros((16,), jnp.int32)
        n = jnp.sum(jnp.where(iota16 == lane, cntv[pl.ds(g, 16)], zi))
        start = jnp.sum(jnp.where(iota16 == lane, offv[pl.ds(g, 16)], zi))
        nb = lax.shift_right_logical(n + K - 1, 7)

        def batch_body(bi, bcarry):
            for j in range(K // 16):
                k16 = bi * K + j * 16
                v = packed[pl.ds(start + k16, 16)]
                lanes = k16 + iota16
                valid = lanes < n
                eid = jnp.where(
                    valid, jnp.bitwise_and(v, (1 << EID_BITS) - 1),
                    jnp.bitwise_and(s * 16 + lanes, E_EDGES - 1))
                posl = jnp.where(
                    valid, lax.shift_right_logical(v, EID_BITS),
                    CHUNK + s)
                gidx[pl.ds(j * 16, 16)] = eid
                sidx[pl.ds(j * 16, 16)] = posl
            pltpu.sync_copy(ea_hbm.at[gidx], grow)
            pltpu.sync_copy(grow, chunk_sp.at[sidx], add=True)
            return bcarry

        lax.fori_loop(0, nb, batch_body, 0)
        plsc.subcore_barrier()
        pltpu.sync_copy(chunk_sp.at[pl.ds(myr, CPT)],
                        out_hbm.at[pl.ds(row0 + myr, CPT)])
        return carry

    lax.fori_loop(0, NCH, chunk_body, 0)


def _sc_scatter(edge_index, edge_attr):
    mesh = plsc.VectorSubcoreMesh(core_axis_name="c", subcore_axis_name="s")
    return pl.kernel(
        _sc_body,
        out_type=jax.ShapeDtypeStruct((TOTAL_PAIRS, 2 * OUT), jnp.float32),
        mesh=mesh,
        compiler_params=pltpu.CompilerParams(needs_layout_passes=False),
        scratch_types=[
            pltpu.VMEM((SBLK,), jnp.int32),
            pltpu.VMEM((SBLK,), jnp.int32),
            pltpu.VMEM((EPT,), jnp.int32),
            pltpu.VMEM((NCH,), jnp.int32),
            pltpu.VMEM((NCH,), jnp.int32),
            pltpu.VMEM((NCH,), jnp.int32),
            pltpu.VMEM((K,), jnp.int32),
            pltpu.VMEM((K,), jnp.int32),
            pltpu.VMEM((K, 2 * OUT), jnp.float32),
            pltpu.VMEM((64, 2 * OUT), jnp.float32),
            pltpu.VMEM_SHARED((CHUNK + 16, 2 * OUT), jnp.float32),
        ],
    )(edge_index[0], edge_index[1],
      jnp.pad(edge_attr, ((0, 0), (0, OUT))))


def _full_out_idx():
    off = jnp.repeat(jnp.arange(N_GRAPHS, dtype=jnp.int32) * N_PER, N_PER * N_PER)
    ii = jnp.tile(jnp.repeat(jnp.arange(N_PER, dtype=jnp.int32), N_PER), N_GRAPHS)
    jj = jnp.tile(jnp.arange(N_PER, dtype=jnp.int32), N_GRAPHS * N_PER)
    return jnp.stack([off + ii, off + jj])


def kernel(gm_val, gm_index, edge_index, edge_attr, batch, W):
    delta = _sc_scatter(edge_index, edge_attr)
    out_val = _matmul_add(gm_val, W, delta)
    return _full_out_idx(), out_val


# speedup vs baseline: 1.4523x; 1.4523x over previous
import jax
import jax.numpy as jnp
from jax import lax
from jax.experimental import pallas as pl
from jax.experimental.pallas import tpu as pltpu
from jax.experimental.pallas import tpu_sc as plsc

N_GRAPHS = 256
N_PER = 64
EMB = 16
OUT = 64
E_EDGES = 262144
TOTAL_PAIRS = N_GRAPHS * N_PER * N_PER

HALF = TOTAL_PAIRS // 2
CHUNK = 8192
CH_SH = 13
NCH = HALF // CHUNK
NTILES = 16
EPT = E_EDGES // NTILES
SBLK = 4096
NBLK = EPT // SBLK
K = 128
CPT = CHUNK // NTILES
EID_BITS = 18

ROW_BLK = 8192


def _mm_body(xt_ref, w_ref, d_ref, o_ref):
    mm = lax.dot_general(
        w_ref[...], xt_ref[...],
        dimension_numbers=(((1,), (0,)), ((), ())),
        preferred_element_type=jnp.float32)
    eye = (lax.broadcasted_iota(jnp.int32, (OUT, OUT), 0) ==
           lax.broadcasted_iota(jnp.int32, (OUT, OUT), 1)).astype(jnp.float32)
    dt = lax.dot_general(
        eye, d_ref[:, :OUT],
        dimension_numbers=(((1,), (1,)), ((), ())),
        preferred_element_type=jnp.float32)
    o_ref[...] = mm + dt


def _matmul_add(gm_val, W, delta):
    gmt = jnp.transpose(gm_val)
    grid = TOTAL_PAIRS // ROW_BLK
    outt = pl.pallas_call(
        _mm_body,
        grid=(grid,),
        in_specs=[
            pl.BlockSpec((EMB, ROW_BLK), lambda i: (0, i)),
            pl.BlockSpec((OUT, EMB), lambda i: (0, 0)),
            pl.BlockSpec((ROW_BLK, 2 * OUT), lambda i: (i, 0)),
        ],
        out_specs=pl.BlockSpec((OUT, ROW_BLK), lambda i: (0, i)),
        out_shape=jax.ShapeDtypeStruct((OUT, TOTAL_PAIRS), jnp.float32),
    )(gmt, W, delta)
    return jnp.transpose(outt)


def _sc_body(es_hbm, ed_hbm, ea_hbm, out_hbm,
             sst, sdt, packed,
             cntv, offv, nextv,
             gidx, sidx, grow, zbuf, chunk_sp):
    c = lax.axis_index("c")
    s = lax.axis_index("s")
    lo = c * HALF
    base_e = s * EPT
    iota16 = lax.iota(jnp.int32, 16)

    zero16 = jnp.zeros((16,), jnp.int32)
    for j in range(NCH // 16):
        cntv[pl.ds(j * 16, 16)] = zero16

    def p1_blk(t, carry):
        pltpu.sync_copy(es_hbm.at[pl.ds(base_e + t * SBLK, SBLK)], sst)
        pltpu.sync_copy(ed_hbm.at[pl.ds(base_e + t * SBLK, SBLK)], sdt)

        def p1(i, icarry):
            sv = sst[pl.ds(i * 16, 16)]
            dv = sdt[pl.ds(i * 16, 16)]
            p = sv * OUT + jnp.bitwise_and(dv, N_PER - 1)
            q = p - lo
            mine = (q >= 0) & (q < HALF)
            b = jnp.where(mine, lax.shift_right_logical(q, CH_SH), 0)
            cnt, last = plsc.scan_count(b, mask=mine)
            plsc.addupdate_scatter(cntv, [b], cnt, mask=last & mine)
            return icarry

        lax.fori_loop(0, SBLK // 16, p1, 0)
        return carry

    lax.fori_loop(0, NBLK, p1_blk, 0)

    carry = jnp.int32(0)
    for j in range(NCH // 16):
        v = cntv[pl.ds(j * 16, 16)]
        cs = plsc.cumsum(v)
        excl = cs - v + carry
        offv[pl.ds(j * 16, 16)] = excl
        nextv[pl.ds(j * 16, 16)] = excl
        carry = carry + jnp.sum(v)

    def p3_blk(t, carry):
        pltpu.sync_copy(es_hbm.at[pl.ds(base_e + t * SBLK, SBLK)], sst)
        pltpu.sync_copy(ed_hbm.at[pl.ds(base_e + t * SBLK, SBLK)], sdt)

        def p3(i, icarry):
            sv = sst[pl.ds(i * 16, 16)]
            dv = sdt[pl.ds(i * 16, 16)]
            p = sv * OUT + jnp.bitwise_and(dv, N_PER - 1)
            q = p - lo
            mine = (q >= 0) & (q < HALF)
            b = jnp.where(mine, lax.shift_right_logical(q, CH_SH), 0)
            cnt, last = plsc.scan_count(b, mask=mine)
            basev = plsc.load_gather(nextv, [b], mask=mine)
            dest = basev + cnt - 1
            posl = jnp.bitwise_and(q, CHUNK - 1)
            eid = base_e + t * SBLK + i * 16 + iota16
            val = jnp.bitwise_or(lax.shift_left(posl, EID_BITS), eid)
            plsc.store_scatter(packed, [dest], val, mask=mine)
            plsc.addupdate_scatter(nextv, [b], cnt, mask=last & mine)
            return icarry

        lax.fori_loop(0, SBLK // 16, p3, 0)
        return carry

    lax.fori_loop(0, NBLK, p3_blk, 0)

    zf32 = jnp.zeros((16,), jnp.float32)

    def zfill(i, carry):
        for j in range(8):
            zbuf[i, pl.ds(j * 16, 16)] = zf32
        return carry

    lax.fori_loop(0, 64, zfill, 0)

    def chunk_body(ch, carry):
        row0 = lo + ch * CHUNK
        myr = s * CPT
        for z in range(CPT // 64):
            pltpu.sync_copy(zbuf, chunk_sp.at[pl.ds(myr + z * 64, 64)])
        plsc.subcore_barrier()
        g = lax.shift_left(lax.shift_right_logical(ch, 4), 4)
        lane = jnp.bitwise_and(ch, 15)
        zi = jnp.zeros((16,), jnp.int32)
        n = jnp.sum(jnp.where(iota16 == lane, cntv[pl.ds(g, 16)], zi))
        start = jnp.sum(jnp.where(iota16 == lane, offv[pl.ds(g, 16)], zi))
        nb = lax.shift_right_logical(n + K - 1, 7)

        def batch_body(bi, bcarry):
            for j in range(K // 16):
                k16 = bi * K + j * 16
                v = packed[pl.ds(start + k16, 16)]
                lanes = k16 + iota16
                valid = lanes < n
                eid = jnp.where(
                    valid, jnp.bitwise_and(v, (1 << EID_BITS) - 1),
                    jnp.bitwise_and(s * 16 + lanes, E_EDGES - 1))
                posl = jnp.where(
                    valid, lax.shift_right_logical(v, EID_BITS),
                    CHUNK + s)
                gidx[pl.ds(j * 16, 16)] = eid
                sidx[pl.ds(j * 16, 16)] = posl
            pltpu.sync_copy(ea_hbm.at[gidx], grow)
            pltpu.sync_copy(grow, chunk_sp.at[sidx], add=True)
            return bcarry

        lax.fori_loop(0, nb, batch_body, 0)
        plsc.subcore_barrier()
        pltpu.sync_copy(chunk_sp.at[pl.ds(myr, CPT)],
                        out_hbm.at[pl.ds(row0 + myr, CPT)])
        return carry

    lax.fori_loop(0, NCH, chunk_body, 0)


def _sc_scatter(edge_index, edge_attr):
    mesh = plsc.VectorSubcoreMesh(core_axis_name="c", subcore_axis_name="s")
    return pl.kernel(
        _sc_body,
        out_type=jax.ShapeDtypeStruct((TOTAL_PAIRS, 2 * OUT), jnp.float32),
        mesh=mesh,
        compiler_params=pltpu.CompilerParams(needs_layout_passes=False),
        scratch_types=[
            pltpu.VMEM((SBLK,), jnp.int32),
            pltpu.VMEM((SBLK,), jnp.int32),
            pltpu.VMEM((EPT,), jnp.int32),
            pltpu.VMEM((NCH,), jnp.int32),
            pltpu.VMEM((NCH,), jnp.int32),
            pltpu.VMEM((NCH,), jnp.int32),
            pltpu.VMEM((K,), jnp.int32),
            pltpu.VMEM((K,), jnp.int32),
            pltpu.VMEM((K, 2 * OUT), jnp.float32),
            pltpu.VMEM((64, 2 * OUT), jnp.float32),
            pltpu.VMEM_SHARED((CHUNK + 16, 2 * OUT), jnp.float32),
        ],
    )(edge_index[0], edge_index[1],
      jnp.pad(edge_attr, ((0, 0), (0, OUT))))


def _full_out_idx():
    off = jnp.repeat(jnp.arange(N_GRAPHS, dtype=jnp.int32) * N_PER, N_PER * N_PER)
    ii = jnp.tile(jnp.repeat(jnp.arange(N_PER, dtype=jnp.int32), N_PER), N_GRAPHS)
    jj = jnp.tile(jnp.arange(N_PER, dtype=jnp.int32), N_GRAPHS * N_PER)
    return jnp.stack([off + ii, off + jj])


def kernel(gm_val, gm_index, edge_index, edge_attr, batch, W):
    delta = _sc_scatter(edge_index, edge_attr)
    out_val = _matmul_add(gm_val, W, delta)
    return _full_out_idx(), out_val
